# baseline (speedup 1.0000x reference)
import functools

import jax
import jax.numpy as jnp
from jax import lax
from jax.experimental import pallas as pl
from jax.experimental.pallas import tpu as pltpu
from jax.experimental.pallas import tpu_sc as plsc

NDRUG = 10000
NDIS = 2000
DIM = 128
NE = 160000
NB = 4096
WR = 0.5
WD = 0.5
POS_W = 5.0
INV_T = 20.0

NC = 2
NS = 16
NW = NC * NS
CHUNK = 32
EPW = 10240
NCH = EPW // CHUNK
NHALF = 8
HCH = NCH // NHALF
HEP = EPW // NHALF
HPAIR = HCH // 2
EPAD = NS * EPW

RPAD = 10240
DPAD = 2048
RSTRIPE = RPAD // NS
DSTRIPE = DPAD // NS
BPW = NB // NW


def _sc_mesh():
    return plsc.VectorSubcoreMesh(core_axis_name="c", subcore_axis_name="s")


def _edge_pass_body(src_hbm, dst_hbm, val_hbm, er_hbm, ed_hbm, zeros_hbm,
                    zr_hbm, zd_hbm,
                    acc_r, acc_d, gidx_v, sidx_v, val_v,
                    gbuf0, gbuf1, sbuf0, sbuf1,
                    sem_g0, sem_g1, sem_s0, sem_s1):
    c = lax.axis_index("c")
    s = lax.axis_index("s")

    def scale(sbuf, gbuf, j):
        @plsc.parallel_loop(0, CHUNK, 1, unroll=8)
        def sbody(k):
            vs = plsc.load_gather(
                val_v, [jnp.broadcast_to(j * CHUNK + k, (16,))])
            for jj in range(DIM // 16):
                sl = pl.ds(jj * 16, 16)
                sbuf[k, sl] = gbuf[k, sl] * vs

    def run_dir(acc, tab_spm, tab_hbm, tstripe, gidx_hbm, sidx_hbm,
                out_hbm, stripe):
        pltpu.sync_copy(zeros_hbm.at[pl.ds(0, stripe)],
                        acc.at[pl.ds(s * stripe, stripe)])
        pltpu.sync_copy(tab_hbm.at[pl.ds(s * tstripe, tstripe)],
                        tab_spm.at[pl.ds(s * tstripe, tstripe)])
        plsc.subcore_barrier()

        gbufs = (gbuf0, gbuf1)
        sbufs = (sbuf0, sbuf1)
        gsems = (sem_g0, sem_g1)
        ssems = (sem_s0, sem_s1)

        for h in range(NHALF):
            pltpu.sync_copy(gidx_hbm.at[pl.ds(s * NCH + h * HCH, HCH)],
                            gidx_v)
            pltpu.sync_copy(sidx_hbm.at[pl.ds(s * NCH + h * HCH, HCH)],
                            sidx_v)
            pltpu.sync_copy(val_hbm.at[pl.ds(s * EPW + h * HEP, HEP)], val_v)

            pltpu.async_copy(tab_spm.at[gidx_v.at[0]], gbuf0, sem_g0)
            pltpu.async_copy(tab_spm.at[gidx_v.at[1]], gbuf1, sem_g1)

            def pair(i, carry):
                for b in range(2):
                    j = 2 * i + b
                    pltpu.make_async_copy(tab_spm.at[gidx_v.at[j]],
                                          gbufs[b], gsems[b]).wait()

                    @pl.when(i > 0)
                    def _():
                        pltpu.make_async_copy(sbufs[b],
                                              acc.at[sidx_v.at[j]],
                                              ssems[b]).wait()

                    scale(sbufs[b], gbufs[b], j)

                    @pl.when(i < HPAIR - 1)
                    def _():
                        pltpu.async_copy(tab_spm.at[gidx_v.at[j + 2]],
                                         gbufs[b], gsems[b])

                    pltpu.async_copy(sbufs[b], acc.at[sidx_v.at[j]],
                                     ssems[b], add=True)
                return carry

            lax.fori_loop(0, HPAIR, pair, 0)
            pltpu.make_async_copy(sbuf0, acc.at[sidx_v.at[0]], sem_s0).wait()
            pltpu.make_async_copy(sbuf1, acc.at[sidx_v.at[0]], sem_s1).wait()

        plsc.subcore_barrier()
        pltpu.sync_copy(acc.at[pl.ds(s * stripe, stripe)],
                        out_hbm.at[pl.ds(s * stripe, stripe)])

    @pl.when(c == 0)
    def _():
        run_dir(acc_r, acc_d, ed_hbm, DSTRIPE, dst_hbm, src_hbm,
                zr_hbm, RSTRIPE)

    @pl.when(c == 1)
    def _():
        run_dir(acc_d, acc_r, er_hbm, RSTRIPE, src_hbm, dst_hbm,
                zd_hbm, DSTRIPE)


def _edge_pass(src_p, dst_p, val_p, er_tab, ed_tab, zeros_rows):
    f = pl.kernel(
        _edge_pass_body,
        out_type=[
            jax.ShapeDtypeStruct((RPAD, DIM), jnp.float32),
            jax.ShapeDtypeStruct((DPAD, DIM), jnp.float32),
        ],
        mesh=_sc_mesh(),
        scratch_types=[
            pltpu.VMEM_SHARED((RPAD, DIM), jnp.float32),
            pltpu.VMEM_SHARED((DPAD, DIM), jnp.float32),
            pltpu.VMEM((HCH, CHUNK), jnp.int32),
            pltpu.VMEM((HCH, CHUNK), jnp.int32),
            pltpu.VMEM((HEP,), jnp.float32),
            pltpu.VMEM((CHUNK, DIM), jnp.float32),
            pltpu.VMEM((CHUNK, DIM), jnp.float32),
            pltpu.VMEM((CHUNK, DIM), jnp.float32),
            pltpu.VMEM((CHUNK, DIM), jnp.float32),
            pltpu.SemaphoreType.DMA,
            pltpu.SemaphoreType.DMA,
            pltpu.SemaphoreType.DMA,
            pltpu.SemaphoreType.DMA,
        ],
        compiler_params=pltpu.CompilerParams(needs_layout_passes=False),
    )
    return f(src_p, dst_p, val_p, er_tab, ed_tab, zeros_rows)


def _gather_body(gr_hbm, gd_hbm, er0_hbm, zr2_hbm, ed0_hbm, zd2_hbm,
                 drugs_hbm, dis_hbm,
                 erb_hbm, grb_hbm, edb_hbm, gdb_hbm, vfr_hbm, vfd_hbm,
                 idx_v, buf, buf2, all_v, dom_v, vf_v, sem, sem2):
    c = lax.axis_index("c")
    s = lax.axis_index("s")
    w = c * NS + s
    b0 = w * BPW

    def addbuf():
        @plsc.parallel_loop(0, BPW, 1, unroll=4)
        def abody(k):
            for jj in range(DIM // 16):
                sl = pl.ds(jj * 16, 16)
                buf[k, sl] = buf[k, sl] + buf2[k, sl]

    pltpu.sync_copy(drugs_hbm.at[pl.ds(b0, BPW)], idx_v)
    d1 = pltpu.async_copy(er0_hbm.at[idx_v], buf, sem)
    d2 = pltpu.async_copy(zr2_hbm.at[idx_v], buf2, sem2)
    d1.wait()
    d2.wait()
    addbuf()
    pltpu.sync_copy(buf, erb_hbm.at[pl.ds(b0, BPW)])
    pltpu.async_copy(gr_hbm.at[idx_v], buf, sem).wait()
    pltpu.sync_copy(buf, grb_hbm.at[pl.ds(b0, BPW)])

    pltpu.sync_copy(dis_hbm.at[pl.ds(b0, BPW)], idx_v)
    d1 = pltpu.async_copy(ed0_hbm.at[idx_v], buf, sem)
    d2 = pltpu.async_copy(zd2_hbm.at[idx_v], buf2, sem2)
    d1.wait()
    d2.wait()
    addbuf()
    pltpu.sync_copy(buf, edb_hbm.at[pl.ds(b0, BPW)])
    pltpu.async_copy(gd_hbm.at[idx_v], buf, sem).wait()
    pltpu.sync_copy(buf, gdb_hbm.at[pl.ds(b0, BPW)])

    def dedup(index_hbm, out_hbm):
        pltpu.sync_copy(index_hbm, all_v)

        def scat(i, carry):
            sl = pl.ds(i * 16, 16)
            idx = all_v[sl]
            bvec = lax.iota(jnp.int32, 16) + i * 16
            plsc.store_scatter(dom_v, [idx], bvec)
            return carry

        lax.fori_loop(0, NB // 16, scat, 0)

        def gath(i, carry):
            sl = pl.ds(i * 16, 16)
            idx = all_v[sl]
            bvec = lax.iota(jnp.int32, 16) + i * 16
            got = plsc.load_gather(dom_v, [idx])
            vf_v[sl] = jnp.where(got == bvec, 1.0, 0.0).astype(jnp.float32)
            return carry

        lax.fori_loop(0, NB // 16, gath, 0)
        pltpu.sync_copy(vf_v, out_hbm)

    @pl.when(w == 0)
    def _():
        dedup(drugs_hbm, vfr_hbm)

    @pl.when(w == 1)
    def _():
        dedup(dis_hbm, vfd_hbm)


def _gather_batch(g_r, g_d, er0, zr2, ed0, zd2, drugs, diseases):
    f = pl.kernel(
        _gather_body,
        out_type=[
            jax.ShapeDtypeStruct((NB, DIM), jnp.float32),
            jax.ShapeDtypeStruct((NB, DIM), jnp.float32),
            jax.ShapeDtypeStruct((NB, DIM), jnp.float32),
            jax.ShapeDtypeStruct((NB, DIM), jnp.float32),
            jax.ShapeDtypeStruct((NB,), jnp.float32),
            jax.ShapeDtypeStruct((NB,), jnp.float32),
        ],
        mesh=_sc_mesh(),
        scratch_types=[
            pltpu.VMEM((BPW,), jnp.int32),
            pltpu.VMEM((BPW, DIM), jnp.float32),
            pltpu.VMEM((BPW, DIM), jnp.float32),
            pltpu.VMEM((NB,), jnp.int32),
            pltpu.VMEM((RPAD,), jnp.int32),
            pltpu.VMEM((NB,), jnp.float32),
            pltpu.SemaphoreType.DMA,
            pltpu.SemaphoreType.DMA,
        ],
        compiler_params=pltpu.CompilerParams(needs_layout_passes=False),
    )
    return f(g_r, g_d, er0, zr2, ed0, zd2, drugs, diseases)


_C1_BR = 1000


def _c1_body(er0_ref, zr1_ref, ed0_ref, zd1_ref, y_ref, x_ref):
    y_ref[...] = er0_ref[...] + zr1_ref[...]

    @pl.when(pl.program_id(0) == 0)
    def _():
        x_ref[...] = ed0_ref[...] + zd1_ref[...]


def _combine1(E_r_0, E_d_0, zr1p, zd1p):
    n = NDRUG // _C1_BR
    return pl.pallas_call(
        _c1_body,
        grid=(n,),
        in_specs=[
            pl.BlockSpec((_C1_BR, DIM), lambda i: (i, 0)),
            pl.BlockSpec((_C1_BR, DIM), lambda i: (i, 0)),
            pl.BlockSpec((NDIS, DIM), lambda i: (0, 0)),
            pl.BlockSpec((NDIS, DIM), lambda i: (0, 0)),
        ],
        out_specs=[
            pl.BlockSpec((_C1_BR, DIM), lambda i: (i, 0)),
            pl.BlockSpec((NDIS, DIM), lambda i: (0, 0)),
        ],
        out_shape=[
            jax.ShapeDtypeStruct((RPAD, DIM), jnp.float32),
            jax.ShapeDtypeStruct((DPAD, DIM), jnp.float32),
        ],
    )(E_r_0, zr1p, E_d_0, zd1p)


_M_BR = 400
_M_STEPS = NDRUG // _M_BR


def _m_body(rec_ref, x_ref, y_ref, er0_ref, ed0_ref,
            gr_ref, gd_ref, qacc_ref):
    i = pl.program_id(0)

    @pl.when(i == 0)
    def _():
        qacc_ref[...] = jnp.zeros_like(qacc_ref)

    rec_blk = rec_ref[...]
    gr_ref[...] = er0_ref[...] + jnp.dot(
        rec_blk, x_ref[...], preferred_element_type=jnp.float32)
    qacc_ref[...] += lax.dot_general(
        rec_blk, y_ref[...], (((0,), (0,)), ((), ())),
        preferred_element_type=jnp.float32)

    @pl.when(i == _M_STEPS - 1)
    def _():
        gd_ref[...] = ed0_ref[...] + qacc_ref[...]


def _matmul_pass(rec, X, Y, E_r_0, E_d_0):
    return pl.pallas_call(
        _m_body,
        grid=(_M_STEPS,),
        in_specs=[
            pl.BlockSpec((_M_BR, NDIS), lambda i: (i, 0)),
            pl.BlockSpec((NDIS, DIM), lambda i: (0, 0)),
            pl.BlockSpec((_M_BR, DIM), lambda i: (i, 0)),
            pl.BlockSpec((_M_BR, DIM), lambda i: (i, 0)),
            pl.BlockSpec((NDIS, DIM), lambda i: (0, 0)),
        ],
        out_specs=[
            pl.BlockSpec((_M_BR, DIM), lambda i: (i, 0)),
            pl.BlockSpec((NDIS, DIM), lambda i: (0, 0)),
        ],
        out_shape=[
            jax.ShapeDtypeStruct((NDRUG, DIM), jnp.float32),
            jax.ShapeDtypeStruct((NDIS, DIM), jnp.float32),
        ],
        scratch_shapes=[pltpu.VMEM((NDIS, DIM), jnp.float32)],
    )(rec, X, Y, E_r_0, E_d_0)


def _combine2(E_r_0, E_d_0, zr2p, zd2p):
    n = NDRUG // _C1_BR
    return pl.pallas_call(
        _c1_body,
        grid=(n,),
        in_specs=[
            pl.BlockSpec((_C1_BR, DIM), lambda i: (i, 0)),
            pl.BlockSpec((_C1_BR, DIM), lambda i: (i, 0)),
            pl.BlockSpec((NDIS, DIM), lambda i: (0, 0)),
            pl.BlockSpec((NDIS, DIM), lambda i: (0, 0)),
        ],
        out_specs=[
            pl.BlockSpec((_C1_BR, DIM), lambda i: (i, 0)),
            pl.BlockSpec((NDIS, DIM), lambda i: (0, 0)),
        ],
        out_shape=[
            jax.ShapeDtypeStruct((NDRUG, DIM), jnp.float32),
            jax.ShapeDtypeStruct((NDIS, DIM), jnp.float32),
        ],
    )(E_r_0, zr2p, E_d_0, zd2p)


_S_BR = 512
_S_STEPS = NB // _S_BR


def _norm_rows(x):
    n = jnp.sqrt(jnp.sum(x * x, axis=1, keepdims=True))
    return x / jnp.maximum(n, 1e-12)


def _s_body(erb_ref, grb_ref, edb_ref, gdb_ref, grf_ref, gdf_ref,
            vfr_ref, vfd_ref, labels_ref,
            p_ref, loss_ref, n2r_ref, n2d_ref, acc_ref):
    i = pl.program_id(0)
    r0 = i * _S_BR

    @pl.when(i == 0)
    def _():
        n2r_ref[...] = _norm_rows(grf_ref[...])
        n2d_ref[...] = _norm_rows(gdf_ref[...])
        acc_ref[0] = 0.0
        acc_ref[1] = 0.0
        acc_ref[2] = 0.0

    vfr_row = vfr_ref[0, :]
    vfd_row = vfd_ref[0, :]
    vfr_blk = vfr_ref[0, pl.ds(r0, _S_BR)]
    vfd_blk = vfd_ref[0, pl.ds(r0, _S_BR)]

    n1r = _norm_rows(erb_ref[...])
    n2r_blk = n2r_ref[pl.ds(r0, _S_BR), :]
    pos_r = jnp.sum(n1r * n2r_blk, axis=1)
    s_r = lax.dot_general(n1r, n2r_ref[...], (((1,), (1,)), ((), ())),
                          preferred_element_type=jnp.float32)
    all_r = jnp.sum(jnp.exp(s_r * INV_T) * vfr_row[None, :], axis=1)
    acc_ref[0] += jnp.sum(vfr_blk * (jnp.log(all_r) - pos_r * INV_T))

    n1d = _norm_rows(edb_ref[...])
    n2d_blk = n2d_ref[pl.ds(r0, _S_BR), :]
    pos_d = jnp.sum(n1d * n2d_blk, axis=1)
    s_d = lax.dot_general(n1d, n2d_ref[...], (((1,), (1,)), ((), ())),
                          preferred_element_type=jnp.float32)
    all_d = jnp.sum(jnp.exp(s_d * INV_T) * vfd_row[None, :], axis=1)
    acc_ref[1] += jnp.sum(vfd_blk * (jnp.log(all_d) - pos_d * INV_T))

    demb = erb_ref[...] + grb_ref[...]
    hemb = edb_ref[...] + gdb_ref[...]
    scores = (WR * WD) * jnp.sum(demb * hemb, axis=1)
    p = jax.nn.sigmoid(scores)
    p_ref[0, :] = p
    pc = jnp.clip(p, 1e-7, 1.0 - 1e-7)
    lab = labels_ref[0, pl.ds(r0, _S_BR)]
    wgt = POS_W * lab + 1.0 - lab
    acc_ref[2] += jnp.sum(
        wgt * -(lab * jnp.log(pc) + (1.0 - lab) * jnp.log(1.0 - pc)))

    @pl.when(i == _S_STEPS - 1)
    def _():
        nvr = jnp.sum(vfr_row)
        nvd = jnp.sum(vfd_row)
        ssl = 0.05 * (acc_ref[0] / nvr) + 0.05 * (acc_ref[1] / nvd)
        loss_ref[...] = jnp.full((1, 1), acc_ref[2] / NB + 0.3 * ssl,
                                 jnp.float32)


def _ssl_bce(er_b, gr_b, ed_b, gd_b, vfr, vfd, labels):
    return pl.pallas_call(
        _s_body,
        grid=(_S_STEPS,),
        in_specs=[
            pl.BlockSpec((_S_BR, DIM), lambda i: (i, 0)),
            pl.BlockSpec((_S_BR, DIM), lambda i: (i, 0)),
            pl.BlockSpec((_S_BR, DIM), lambda i: (i, 0)),
            pl.BlockSpec((_S_BR, DIM), lambda i: (i, 0)),
            pl.BlockSpec((NB, DIM), lambda i: (0, 0)),
            pl.BlockSpec((NB, DIM), lambda i: (0, 0)),
            pl.BlockSpec((1, NB), lambda i: (0, 0)),
            pl.BlockSpec((1, NB), lambda i: (0, 0)),
            pl.BlockSpec((1, NB), lambda i: (0, 0)),
        ],
        out_specs=[
            pl.BlockSpec((1, _S_BR), lambda i: (0, i)),
            pl.BlockSpec((1, 1), lambda i: (0, 0)),
        ],
        out_shape=[
            jax.ShapeDtypeStruct((1, NB), jnp.float32),
            jax.ShapeDtypeStruct((1, 1), jnp.float32),
        ],
        scratch_shapes=[
            pltpu.VMEM((NB, DIM), jnp.float32),
            pltpu.VMEM((NB, DIM), jnp.float32),
            pltpu.SMEM((4,), jnp.float32),
        ],
    )(er_b, gr_b, ed_b, gd_b, gr_b, gd_b, vfr, vfd, labels)


def kernel(drugs, diseases, labels, E_r_0, E_d_0, adj_idx, adj_val, rec):
    src = adj_idx[0]
    dst = adj_idx[1]
    padn = EPAD - NE
    src_p = jnp.concatenate([src, jnp.zeros((padn,), jnp.int32)])
    dst_p = jnp.concatenate([dst, jnp.zeros((padn,), jnp.int32)])
    val_p = jnp.concatenate([adj_val, jnp.zeros((padn,), jnp.float32)])
    src_p = src_p.reshape(NS * NCH, CHUNK)
    dst_p = dst_p.reshape(NS * NCH, CHUNK)
    zeros_rows = jnp.zeros((RSTRIPE, DIM), jnp.float32)
    Er0p = jnp.concatenate(
        [E_r_0, jnp.zeros((RPAD - NDRUG, DIM), jnp.float32)])
    Ed0p = jnp.concatenate(
        [E_d_0, jnp.zeros((DPAD - NDIS, DIM), jnp.float32)])

    zr1p, zd1p = _edge_pass(src_p, dst_p, val_p, Er0p, Ed0p, zeros_rows)
    Y, X = _combine1(E_r_0, E_d_0, zr1p, zd1p)

    zr2p, zd2p = _edge_pass(src_p, dst_p, val_p, Y, X, zeros_rows)
    G_r, G_d = _matmul_pass(rec, X, Y, E_r_0, E_d_0)

    er_b, gr_b, ed_b, gd_b, vfr, vfd = _gather_batch(
        G_r, G_d, E_r_0, zr2p, E_d_0, zd2p, drugs, diseases)

    p2, loss2 = _ssl_bce(er_b, gr_b, ed_b, gd_b,
                         vfr.reshape(1, NB), vfd.reshape(1, NB),
                         labels.reshape(1, NB))
    return (loss2[0, 0], p2.reshape(NB))


# speedup vs baseline: 1.0390x; 1.0390x over previous
import functools

import jax
import jax.numpy as jnp
from jax import lax
from jax.experimental import pallas as pl
from jax.experimental.pallas import tpu as pltpu
from jax.experimental.pallas import tpu_sc as plsc

NDRUG = 10000
NDIS = 2000
DIM = 128
NE = 160000
NB = 4096
WR = 0.5
WD = 0.5
POS_W = 5.0
INV_T = 20.0

NC = 2
NS = 16
NW = NC * NS
CHUNK = 32
EPW = 10240
NCH = EPW // CHUNK
NHALF = 8
HCH = NCH // NHALF
HEP = EPW // NHALF
HPAIR = HCH // 2
EPAD = NS * EPW

RPAD = 10240
DPAD = 2048
RSTRIPE = RPAD // NS
DSTRIPE = DPAD // NS
BPW = NB // NW


def _sc_mesh():
    return plsc.VectorSubcoreMesh(core_axis_name="c", subcore_axis_name="s")


def _edge_pass_body(src_hbm, dst_hbm, val_hbm, er_hbm, ed_hbm, zeros_hbm,
                    zr_hbm, zd_hbm,
                    acc_r, acc_d, gidx_v, sidx_v, val_v,
                    gbuf0, gbuf1, sbuf0, sbuf1,
                    sem_g0, sem_g1, sem_s0, sem_s1):
    c = lax.axis_index("c")
    s = lax.axis_index("s")

    def scale(sbuf, gbuf, j):
        @plsc.parallel_loop(0, CHUNK, 1, unroll=4)
        def sbody(k):
            vs = plsc.load_gather(
                val_v, [jnp.broadcast_to(j * CHUNK + k, (16,))])
            for jj in range(DIM // 16):
                sl = pl.ds(jj * 16, 16)
                sbuf[k, sl] = gbuf[k, sl] * vs

    def run_dir(acc, tab_spm, tab_hbm, tstripe, gidx_hbm, sidx_hbm,
                out_hbm, stripe):
        pltpu.sync_copy(zeros_hbm.at[pl.ds(0, stripe)],
                        acc.at[pl.ds(s * stripe, stripe)])
        pltpu.sync_copy(tab_hbm.at[pl.ds(s * tstripe, tstripe)],
                        tab_spm.at[pl.ds(s * tstripe, tstripe)])
        plsc.subcore_barrier()

        gbufs = (gbuf0, gbuf1)
        sbufs = (sbuf0, sbuf1)
        gsems = (sem_g0, sem_g1)
        ssems = (sem_s0, sem_s1)

        for h in range(NHALF):
            pltpu.sync_copy(gidx_hbm.at[pl.ds(s * NCH + h * HCH, HCH)],
                            gidx_v)
            pltpu.sync_copy(sidx_hbm.at[pl.ds(s * NCH + h * HCH, HCH)],
                            sidx_v)
            pltpu.sync_copy(val_hbm.at[pl.ds(s * EPW + h * HEP, HEP)], val_v)

            pltpu.async_copy(tab_spm.at[gidx_v.at[0]], gbuf0, sem_g0)
            pltpu.async_copy(tab_spm.at[gidx_v.at[1]], gbuf1, sem_g1)

            def pair(i, carry):
                for b in range(2):
                    j = 2 * i + b
                    pltpu.make_async_copy(tab_spm.at[gidx_v.at[j]],
                                          gbufs[b], gsems[b]).wait()

                    @pl.when(i > 0)
                    def _():
                        pltpu.make_async_copy(sbufs[b],
                                              acc.at[sidx_v.at[j]],
                                              ssems[b]).wait()

                    scale(sbufs[b], gbufs[b], j)

                    @pl.when(i < HPAIR - 1)
                    def _():
                        pltpu.async_copy(tab_spm.at[gidx_v.at[j + 2]],
                                         gbufs[b], gsems[b])

                    pltpu.async_copy(sbufs[b], acc.at[sidx_v.at[j]],
                                     ssems[b], add=True)
                return carry

            lax.fori_loop(0, HPAIR, pair, 0)
            pltpu.make_async_copy(sbuf0, acc.at[sidx_v.at[0]], sem_s0).wait()
            pltpu.make_async_copy(sbuf1, acc.at[sidx_v.at[0]], sem_s1).wait()

        plsc.subcore_barrier()
        pltpu.sync_copy(acc.at[pl.ds(s * stripe, stripe)],
                        out_hbm.at[pl.ds(s * stripe, stripe)])

    @pl.when(c == 0)
    def _():
        run_dir(acc_r, acc_d, ed_hbm, DSTRIPE, dst_hbm, src_hbm,
                zr_hbm, RSTRIPE)

    @pl.when(c == 1)
    def _():
        run_dir(acc_d, acc_r, er_hbm, RSTRIPE, src_hbm, dst_hbm,
                zd_hbm, DSTRIPE)


def _edge_pass(src_p, dst_p, val_p, er_tab, ed_tab, zeros_rows):
    f = pl.kernel(
        _edge_pass_body,
        out_type=[
            jax.ShapeDtypeStruct((RPAD, DIM), jnp.float32),
            jax.ShapeDtypeStruct((DPAD, DIM), jnp.float32),
        ],
        mesh=_sc_mesh(),
        scratch_types=[
            pltpu.VMEM_SHARED((RPAD, DIM), jnp.float32),
            pltpu.VMEM_SHARED((DPAD, DIM), jnp.float32),
            pltpu.VMEM((HCH, CHUNK), jnp.int32),
            pltpu.VMEM((HCH, CHUNK), jnp.int32),
            pltpu.VMEM((HEP,), jnp.float32),
            pltpu.VMEM((CHUNK, DIM), jnp.float32),
            pltpu.VMEM((CHUNK, DIM), jnp.float32),
            pltpu.VMEM((CHUNK, DIM), jnp.float32),
            pltpu.VMEM((CHUNK, DIM), jnp.float32),
            pltpu.SemaphoreType.DMA,
            pltpu.SemaphoreType.DMA,
            pltpu.SemaphoreType.DMA,
            pltpu.SemaphoreType.DMA,
        ],
        compiler_params=pltpu.CompilerParams(needs_layout_passes=False),
    )
    return f(src_p, dst_p, val_p, er_tab, ed_tab, zeros_rows)


def _gather_body(gr_hbm, gd_hbm, er0_hbm, zr2_hbm, ed0_hbm, zd2_hbm,
                 drugs_hbm, dis_hbm,
                 erb_hbm, grb_hbm, edb_hbm, gdb_hbm, vfr_hbm, vfd_hbm,
                 idx_v, buf, buf2, all_v, dom_v, vf_v, dmask_v, sem, sem2):
    c = lax.axis_index("c")
    s = lax.axis_index("s")
    w = c * NS + s
    b0 = w * BPW

    def addbuf():
        @plsc.parallel_loop(0, BPW, 1, unroll=4)
        def abody(k):
            for jj in range(DIM // 16):
                sl = pl.ds(jj * 16, 16)
                buf[k, sl] = buf[k, sl] + buf2[k, sl]

    pltpu.sync_copy(drugs_hbm.at[pl.ds(b0, BPW)], idx_v)
    d1 = pltpu.async_copy(er0_hbm.at[idx_v], buf, sem)
    d2 = pltpu.async_copy(zr2_hbm.at[idx_v], buf2, sem2)
    d1.wait()
    d2.wait()
    addbuf()
    pltpu.sync_copy(buf, erb_hbm.at[pl.ds(b0, BPW)])
    pltpu.async_copy(gr_hbm.at[idx_v], buf, sem).wait()
    pltpu.sync_copy(buf, grb_hbm.at[pl.ds(b0, BPW)])

    pltpu.sync_copy(dis_hbm.at[pl.ds(b0, BPW)], idx_v)
    d1 = pltpu.async_copy(ed0_hbm.at[idx_v], buf, sem)
    d2 = pltpu.async_copy(zd2_hbm.at[idx_v], buf2, sem2)
    d1.wait()
    d2.wait()
    addbuf()
    pltpu.sync_copy(buf, edb_hbm.at[pl.ds(b0, BPW)])
    pltpu.async_copy(gd_hbm.at[idx_v], buf, sem).wait()
    pltpu.sync_copy(buf, gdb_hbm.at[pl.ds(b0, BPW)])

    def dedup(index_hbm, out_hbm):
        pltpu.sync_copy(index_hbm, all_v)

        def scat(i, carry):
            sl = pl.ds(i * 16, 16)
            idx = all_v[sl]
            bvec = lax.iota(jnp.int32, 16) + i * 16
            plsc.store_scatter(dom_v, [idx], bvec)
            return carry

        lax.fori_loop(0, NB // 16, scat, 0)

        def gath(i, carry):
            sl = pl.ds(i * 16, 16)
            idx = all_v[sl]
            bvec = lax.iota(jnp.int32, 16) + i * 16
            got = plsc.load_gather(dom_v, [idx])
            vf_v[sl] = jnp.where(got == bvec, 1.0, 0.0).astype(jnp.float32)
            return carry

        lax.fori_loop(0, NB // 16, gath, 0)
        pltpu.sync_copy(vf_v, out_hbm)

    @pl.when(w == 0)
    def _():
        dedup(drugs_hbm, vfr_hbm)

    @pl.when(w == 1)
    def _():
        def zloop(i, carry):
            dmask_v[pl.ds(i * 16, 16)] = jnp.zeros((16,), jnp.float32)
            return carry

        lax.fori_loop(0, DPAD // 16, zloop, 0)
        pltpu.sync_copy(dis_hbm, all_v)

        def mscat(i, carry):
            idx = all_v[pl.ds(i * 16, 16)]
            plsc.store_scatter(dmask_v, [idx], jnp.ones((16,), jnp.float32))
            return carry

        lax.fori_loop(0, NB // 16, mscat, 0)
        pltpu.sync_copy(dmask_v, vfd_hbm)


def _gather_batch(g_r, g_d, er0, zr2, ed0, zd2, drugs, diseases):
    f = pl.kernel(
        _gather_body,
        out_type=[
            jax.ShapeDtypeStruct((NB, DIM), jnp.float32),
            jax.ShapeDtypeStruct((NB, DIM), jnp.float32),
            jax.ShapeDtypeStruct((NB, DIM), jnp.float32),
            jax.ShapeDtypeStruct((NB, DIM), jnp.float32),
            jax.ShapeDtypeStruct((NB,), jnp.float32),
            jax.ShapeDtypeStruct((DPAD,), jnp.float32),
        ],
        mesh=_sc_mesh(),
        scratch_types=[
            pltpu.VMEM((BPW,), jnp.int32),
            pltpu.VMEM((BPW, DIM), jnp.float32),
            pltpu.VMEM((BPW, DIM), jnp.float32),
            pltpu.VMEM((NB,), jnp.int32),
            pltpu.VMEM((RPAD,), jnp.int32),
            pltpu.VMEM((NB,), jnp.float32),
            pltpu.VMEM((DPAD,), jnp.float32),
            pltpu.SemaphoreType.DMA,
            pltpu.SemaphoreType.DMA,
        ],
        compiler_params=pltpu.CompilerParams(needs_layout_passes=False),
    )
    return f(g_r, g_d, er0, zr2, ed0, zd2, drugs, diseases)


_C1_BR = 1000


def _c1_body(er0_ref, zr1_ref, ed0_ref, zd1_ref, y_ref, x_ref):
    y_ref[...] = er0_ref[...] + zr1_ref[...]

    @pl.when(pl.program_id(0) == 0)
    def _():
        x_ref[...] = ed0_ref[...] + zd1_ref[...]


def _combine1(E_r_0, E_d_0, zr1p, zd1p):
    n = NDRUG // _C1_BR
    return pl.pallas_call(
        _c1_body,
        grid=(n,),
        in_specs=[
            pl.BlockSpec((_C1_BR, DIM), lambda i: (i, 0)),
            pl.BlockSpec((_C1_BR, DIM), lambda i: (i, 0)),
            pl.BlockSpec((NDIS, DIM), lambda i: (0, 0)),
            pl.BlockSpec((NDIS, DIM), lambda i: (0, 0)),
        ],
        out_specs=[
            pl.BlockSpec((_C1_BR, DIM), lambda i: (i, 0)),
            pl.BlockSpec((NDIS, DIM), lambda i: (0, 0)),
        ],
        out_shape=[
            jax.ShapeDtypeStruct((RPAD, DIM), jnp.float32),
            jax.ShapeDtypeStruct((DPAD, DIM), jnp.float32),
        ],
    )(E_r_0, zr1p, E_d_0, zd1p)


_M_BR = 400
_M_STEPS = NDRUG // _M_BR


def _m_body(rec_ref, x_ref, y_ref, er0_ref, ed0_ref,
            gr_ref, gd_ref, qacc_ref):
    i = pl.program_id(0)

    @pl.when(i == 0)
    def _():
        qacc_ref[...] = jnp.zeros_like(qacc_ref)

    rec_blk = rec_ref[...]
    gr_ref[...] = er0_ref[...] + jnp.dot(
        rec_blk, x_ref[...], preferred_element_type=jnp.float32)
    qacc_ref[...] += lax.dot_general(
        rec_blk, y_ref[...], (((0,), (0,)), ((), ())),
        preferred_element_type=jnp.float32)

    @pl.when(i == _M_STEPS - 1)
    def _():
        gd_ref[...] = ed0_ref[...] + qacc_ref[...]


def _matmul_pass(rec, X, Y, E_r_0, E_d_0):
    return pl.pallas_call(
        _m_body,
        grid=(_M_STEPS,),
        in_specs=[
            pl.BlockSpec((_M_BR, NDIS), lambda i: (i, 0)),
            pl.BlockSpec((NDIS, DIM), lambda i: (0, 0)),
            pl.BlockSpec((_M_BR, DIM), lambda i: (i, 0)),
            pl.BlockSpec((_M_BR, DIM), lambda i: (i, 0)),
            pl.BlockSpec((NDIS, DIM), lambda i: (0, 0)),
        ],
        out_specs=[
            pl.BlockSpec((_M_BR, DIM), lambda i: (i, 0)),
            pl.BlockSpec((NDIS, DIM), lambda i: (0, 0)),
        ],
        out_shape=[
            jax.ShapeDtypeStruct((NDRUG, DIM), jnp.float32),
            jax.ShapeDtypeStruct((NDIS, DIM), jnp.float32),
        ],
        scratch_shapes=[pltpu.VMEM((NDIS, DIM), jnp.float32)],
    )(rec, X, Y, E_r_0, E_d_0)


def _combine2(E_r_0, E_d_0, zr2p, zd2p):
    n = NDRUG // _C1_BR
    return pl.pallas_call(
        _c1_body,
        grid=(n,),
        in_specs=[
            pl.BlockSpec((_C1_BR, DIM), lambda i: (i, 0)),
            pl.BlockSpec((_C1_BR, DIM), lambda i: (i, 0)),
            pl.BlockSpec((NDIS, DIM), lambda i: (0, 0)),
            pl.BlockSpec((NDIS, DIM), lambda i: (0, 0)),
        ],
        out_specs=[
            pl.BlockSpec((_C1_BR, DIM), lambda i: (i, 0)),
            pl.BlockSpec((NDIS, DIM), lambda i: (0, 0)),
        ],
        out_shape=[
            jax.ShapeDtypeStruct((NDRUG, DIM), jnp.float32),
            jax.ShapeDtypeStruct((NDIS, DIM), jnp.float32),
        ],
    )(E_r_0, zr2p, E_d_0, zd2p)


_S_BR = 256
_S_STEPS = NB // _S_BR


def _norm_rows(x):
    n = jnp.sqrt(jnp.sum(x * x, axis=1, keepdims=True))
    return x / jnp.maximum(n, 1e-12)


_D_BR = 400
_D_STEPS = NDIS // _D_BR


def _s_body(erb_ref, grb_ref, edb_ref, gdb_ref, grf_ref, gddom_ref,
            ed0f_ref, zd2f_ref, vfr_ref, vfdd_ref, vfdc_ref, labels_ref,
            p_ref, loss_ref, n2r_ref, n1d_ref, n2d_ref, acc_ref):
    i = pl.program_id(0)
    r0 = i * _S_BR

    @pl.when(i == 0)
    def _():
        n2r_ref[...] = _norm_rows(grf_ref[...])
        n1d_ref[...] = _norm_rows(ed0f_ref[...] + zd2f_ref[...])
        n2d_ref[...] = _norm_rows(gddom_ref[...])
        acc_ref[0] = 0.0
        acc_ref[1] = 0.0
        acc_ref[2] = 0.0

    vfr_row = vfr_ref[0, :]
    vfr_blk = vfr_ref[0, pl.ds(r0, _S_BR)]

    n1r = _norm_rows(erb_ref[...])
    n2r_blk = n2r_ref[pl.ds(r0, _S_BR), :]
    pos_r = jnp.sum(n1r * n2r_blk, axis=1)
    s_r = lax.dot_general(n1r, n2r_ref[...], (((1,), (1,)), ((), ())),
                          preferred_element_type=jnp.float32)
    all_r = jnp.sum(jnp.exp(s_r * INV_T) * vfr_row[None, :], axis=1)
    acc_ref[0] += jnp.sum(vfr_blk * (jnp.log(all_r) - pos_r * INV_T))

    @pl.when(i < _D_STEPS)
    def _():
        d0 = i * _D_BR
        md_row = vfdd_ref[0, pl.ds(0, NDIS)]
        md_blk = vfdc_ref[pl.ds(d0, _D_BR), 0]
        n1blk = n1d_ref[pl.ds(d0, _D_BR), :]
        pos_d = jnp.sum(n1blk * n2d_ref[pl.ds(d0, _D_BR), :], axis=1)
        s_d = lax.dot_general(n1blk, n2d_ref[...], (((1,), (1,)), ((), ())),
                              preferred_element_type=jnp.float32)
        all_d = jnp.sum(jnp.exp(s_d * INV_T) * md_row[None, :], axis=1)
        acc_ref[1] += jnp.sum(md_blk * (jnp.log(all_d) - pos_d * INV_T))

    demb = erb_ref[...] + grb_ref[...]
    hemb = edb_ref[...] + gdb_ref[...]
    scores = (WR * WD) * jnp.sum(demb * hemb, axis=1)
    p = jax.nn.sigmoid(scores)
    p_ref[0, :] = p
    pc = jnp.clip(p, 1e-7, 1.0 - 1e-7)
    lab = labels_ref[0, pl.ds(r0, _S_BR)]
    wgt = POS_W * lab + 1.0 - lab
    acc_ref[2] += jnp.sum(
        wgt * -(lab * jnp.log(pc) + (1.0 - lab) * jnp.log(1.0 - pc)))

    @pl.when(i == _S_STEPS - 1)
    def _():
        nvr = jnp.sum(vfr_row)
        nvd = jnp.sum(vfdd_ref[0, :])
        ssl = 0.05 * (acc_ref[0] / nvr) + 0.05 * (acc_ref[1] / nvd)
        loss_ref[...] = jnp.full((1, 1), acc_ref[2] / NB + 0.3 * ssl,
                                 jnp.float32)


def _ssl_bce(er_b, gr_b, ed_b, gd_b, G_d, E_d_0, zd2p, vfr, vfdd, vfdc,
             labels):
    return pl.pallas_call(
        _s_body,
        grid=(_S_STEPS,),
        in_specs=[
            pl.BlockSpec((_S_BR, DIM), lambda i: (i, 0)),
            pl.BlockSpec((_S_BR, DIM), lambda i: (i, 0)),
            pl.BlockSpec((_S_BR, DIM), lambda i: (i, 0)),
            pl.BlockSpec((_S_BR, DIM), lambda i: (i, 0)),
            pl.BlockSpec((NB, DIM), lambda i: (0, 0)),
            pl.BlockSpec((NDIS, DIM), lambda i: (0, 0)),
            pl.BlockSpec((NDIS, DIM), lambda i: (0, 0)),
            pl.BlockSpec((NDIS, DIM), lambda i: (0, 0)),
            pl.BlockSpec((1, NB), lambda i: (0, 0)),
            pl.BlockSpec((1, DPAD), lambda i: (0, 0)),
            pl.BlockSpec((DPAD, 1), lambda i: (0, 0)),
            pl.BlockSpec((1, NB), lambda i: (0, 0)),
        ],
        out_specs=[
            pl.BlockSpec((1, _S_BR), lambda i: (0, i)),
            pl.BlockSpec((1, 1), lambda i: (0, 0)),
        ],
        out_shape=[
            jax.ShapeDtypeStruct((1, NB), jnp.float32),
            jax.ShapeDtypeStruct((1, 1), jnp.float32),
        ],
        scratch_shapes=[
            pltpu.VMEM((NB, DIM), jnp.float32),
            pltpu.VMEM((NDIS, DIM), jnp.float32),
            pltpu.VMEM((NDIS, DIM), jnp.float32),
            pltpu.SMEM((4,), jnp.float32),
        ],
    )(er_b, gr_b, ed_b, gd_b, gr_b, G_d, E_d_0, zd2p, vfr, vfdd, vfdc,
      labels)


def kernel(drugs, diseases, labels, E_r_0, E_d_0, adj_idx, adj_val, rec):
    src = adj_idx[0]
    dst = adj_idx[1]
    padn = EPAD - NE
    src_p = jnp.concatenate([src, jnp.zeros((padn,), jnp.int32)])
    dst_p = jnp.concatenate([dst, jnp.zeros((padn,), jnp.int32)])
    val_p = jnp.concatenate([adj_val, jnp.zeros((padn,), jnp.float32)])
    src_p = src_p.reshape(NS * NCH, CHUNK)
    dst_p = dst_p.reshape(NS * NCH, CHUNK)
    zeros_rows = jnp.zeros((RSTRIPE, DIM), jnp.float32)
    Er0p = jnp.concatenate(
        [E_r_0, jnp.zeros((RPAD - NDRUG, DIM), jnp.float32)])
    Ed0p = jnp.concatenate(
        [E_d_0, jnp.zeros((DPAD - NDIS, DIM), jnp.float32)])

    zr1p, zd1p = _edge_pass(src_p, dst_p, val_p, Er0p, Ed0p, zeros_rows)
    Y, X = _combine1(E_r_0, E_d_0, zr1p, zd1p)

    zr2p, zd2p = _edge_pass(src_p, dst_p, val_p, Y, X, zeros_rows)
    G_r, G_d = _matmul_pass(rec, X, Y, E_r_0, E_d_0)

    er_b, gr_b, ed_b, gd_b, vfr, vfd = _gather_batch(
        G_r, G_d, E_r_0, zr2p, E_d_0, zd2p, drugs, diseases)

    p2, loss2 = _ssl_bce(er_b, gr_b, ed_b, gd_b, G_d, E_d_0, zd2p,
                         vfr.reshape(1, NB), vfd.reshape(1, DPAD),
                         vfd.reshape(DPAD, 1), labels.reshape(1, NB))
    return (loss2[0, 0], p2.reshape(NB))


# speedup vs baseline: 1.0404x; 1.0013x over previous
import functools

import jax
import jax.numpy as jnp
from jax import lax
from jax.experimental import pallas as pl
from jax.experimental.pallas import tpu as pltpu
from jax.experimental.pallas import tpu_sc as plsc

NDRUG = 10000
NDIS = 2000
DIM = 128
NE = 160000
NB = 4096
WR = 0.5
WD = 0.5
POS_W = 5.0
INV_T = 20.0

NC = 2
NS = 16
NW = NC * NS
CHUNK = 32
EPW = 10240
NCH = EPW // CHUNK
NHALF = 8
HCH = NCH // NHALF
HEP = EPW // NHALF
HPAIR = HCH // 2
EPAD = NS * EPW

RPAD = 10240
DPAD = 2048
RSTRIPE = RPAD // NS
DSTRIPE = DPAD // NS
BPW = NB // NW


def _sc_mesh():
    return plsc.VectorSubcoreMesh(core_axis_name="c", subcore_axis_name="s")


def _edge_pass_body(src_hbm, dst_hbm, val_hbm, er_hbm, ed_hbm, zeros_hbm,
                    zr_hbm, zd_hbm,
                    acc_r, acc_d, gidx_v, sidx_v, val_v,
                    gbuf0, gbuf1, sbuf0, sbuf1,
                    sem_g0, sem_g1, sem_s0, sem_s1):
    c = lax.axis_index("c")
    s = lax.axis_index("s")

    def scale(sbuf, gbuf, j):
        @plsc.parallel_loop(0, CHUNK, 1, unroll=4)
        def sbody(k):
            vs = plsc.load_gather(
                val_v, [jnp.broadcast_to(j * CHUNK + k, (16,))])
            for jj in range(DIM // 16):
                sl = pl.ds(jj * 16, 16)
                sbuf[k, sl] = gbuf[k, sl] * vs

    def run_dir(acc, tab_spm, tab_hbm, tstripe, gidx_hbm, sidx_hbm,
                out_hbm, stripe):
        pltpu.sync_copy(zeros_hbm.at[pl.ds(0, stripe)],
                        acc.at[pl.ds(s * stripe, stripe)])
        pltpu.sync_copy(tab_hbm.at[pl.ds(s * tstripe, tstripe)],
                        tab_spm.at[pl.ds(s * tstripe, tstripe)])
        plsc.subcore_barrier()

        gbufs = (gbuf0, gbuf1)
        sbufs = (sbuf0, sbuf1)
        gsems = (sem_g0, sem_g1)
        ssems = (sem_s0, sem_s1)

        for h in range(NHALF):
            pltpu.sync_copy(gidx_hbm.at[pl.ds(s * NCH + h * HCH, HCH)],
                            gidx_v)
            pltpu.sync_copy(sidx_hbm.at[pl.ds(s * NCH + h * HCH, HCH)],
                            sidx_v)
            pltpu.sync_copy(val_hbm.at[pl.ds(s * EPW + h * HEP, HEP)], val_v)

            pltpu.async_copy(tab_spm.at[gidx_v.at[0]], gbuf0, sem_g0)
            pltpu.async_copy(tab_spm.at[gidx_v.at[1]], gbuf1, sem_g1)

            def pair(i, carry):
                for b in range(2):
                    j = 2 * i + b
                    pltpu.make_async_copy(tab_spm.at[gidx_v.at[j]],
                                          gbufs[b], gsems[b]).wait()

                    @pl.when(i > 0)
                    def _():
                        pltpu.make_async_copy(sbufs[b],
                                              acc.at[sidx_v.at[j]],
                                              ssems[b]).wait()

                    scale(sbufs[b], gbufs[b], j)

                    @pl.when(i < HPAIR - 1)
                    def _():
                        pltpu.async_copy(tab_spm.at[gidx_v.at[j + 2]],
                                         gbufs[b], gsems[b])

                    pltpu.async_copy(sbufs[b], acc.at[sidx_v.at[j]],
                                     ssems[b], add=True)
                return carry

            lax.fori_loop(0, HPAIR, pair, 0)
            pltpu.make_async_copy(sbuf0, acc.at[sidx_v.at[0]], sem_s0).wait()
            pltpu.make_async_copy(sbuf1, acc.at[sidx_v.at[0]], sem_s1).wait()

        plsc.subcore_barrier()
        pltpu.sync_copy(acc.at[pl.ds(s * stripe, stripe)],
                        out_hbm.at[pl.ds(s * stripe, stripe)])

    @pl.when(c == 0)
    def _():
        run_dir(acc_r, acc_d, ed_hbm, DSTRIPE, dst_hbm, src_hbm,
                zr_hbm, RSTRIPE)

    @pl.when(c == 1)
    def _():
        run_dir(acc_d, acc_r, er_hbm, RSTRIPE, src_hbm, dst_hbm,
                zd_hbm, DSTRIPE)


def _edge_pass(src_p, dst_p, val_p, er_tab, ed_tab, zeros_rows):
    f = pl.kernel(
        _edge_pass_body,
        out_type=[
            jax.ShapeDtypeStruct((RPAD, DIM), jnp.float32),
            jax.ShapeDtypeStruct((DPAD, DIM), jnp.float32),
        ],
        mesh=_sc_mesh(),
        scratch_types=[
            pltpu.VMEM_SHARED((RPAD, DIM), jnp.float32),
            pltpu.VMEM_SHARED((DPAD, DIM), jnp.float32),
            pltpu.VMEM((HCH, CHUNK), jnp.int32),
            pltpu.VMEM((HCH, CHUNK), jnp.int32),
            pltpu.VMEM((HEP,), jnp.float32),
            pltpu.VMEM((CHUNK, DIM), jnp.float32),
            pltpu.VMEM((CHUNK, DIM), jnp.float32),
            pltpu.VMEM((CHUNK, DIM), jnp.float32),
            pltpu.VMEM((CHUNK, DIM), jnp.float32),
            pltpu.SemaphoreType.DMA,
            pltpu.SemaphoreType.DMA,
            pltpu.SemaphoreType.DMA,
            pltpu.SemaphoreType.DMA,
        ],
        compiler_params=pltpu.CompilerParams(needs_layout_passes=False),
    )
    return f(src_p, dst_p, val_p, er_tab, ed_tab, zeros_rows)


def _gather_body(gr_hbm, gd_hbm, er0_hbm, zr2_hbm, ed0_hbm, zd2_hbm,
                 drugs_hbm, dis_hbm,
                 erb_hbm, grb_hbm, edb_hbm, gdb_hbm, vfr_hbm, vfd_hbm,
                 idx_v, buf, buf2, all_v, dom_v, vf_v, dmask_v, sem, sem2):
    c = lax.axis_index("c")
    s = lax.axis_index("s")
    w = c * NS + s
    b0 = w * BPW

    def addbuf():
        @plsc.parallel_loop(0, BPW, 1, unroll=4)
        def abody(k):
            for jj in range(DIM // 16):
                sl = pl.ds(jj * 16, 16)
                buf[k, sl] = buf[k, sl] + buf2[k, sl]

    pltpu.sync_copy(drugs_hbm.at[pl.ds(b0, BPW)], idx_v)
    d1 = pltpu.async_copy(er0_hbm.at[idx_v], buf, sem)
    d2 = pltpu.async_copy(zr2_hbm.at[idx_v], buf2, sem2)
    d1.wait()
    d2.wait()
    addbuf()
    pltpu.sync_copy(buf, erb_hbm.at[pl.ds(b0, BPW)])
    pltpu.async_copy(gr_hbm.at[idx_v], buf, sem).wait()
    pltpu.sync_copy(buf, grb_hbm.at[pl.ds(b0, BPW)])

    pltpu.sync_copy(dis_hbm.at[pl.ds(b0, BPW)], idx_v)
    d1 = pltpu.async_copy(ed0_hbm.at[idx_v], buf, sem)
    d2 = pltpu.async_copy(zd2_hbm.at[idx_v], buf2, sem2)
    d1.wait()
    d2.wait()
    addbuf()
    pltpu.sync_copy(buf, edb_hbm.at[pl.ds(b0, BPW)])
    pltpu.async_copy(gd_hbm.at[idx_v], buf, sem).wait()
    pltpu.sync_copy(buf, gdb_hbm.at[pl.ds(b0, BPW)])

    def dedup(index_hbm, out_hbm):
        pltpu.sync_copy(index_hbm, all_v)

        def scat(i, carry):
            sl = pl.ds(i * 16, 16)
            idx = all_v[sl]
            bvec = lax.iota(jnp.int32, 16) + i * 16
            plsc.store_scatter(dom_v, [idx], bvec)
            return carry

        lax.fori_loop(0, NB // 16, scat, 0)

        def gath(i, carry):
            sl = pl.ds(i * 16, 16)
            idx = all_v[sl]
            bvec = lax.iota(jnp.int32, 16) + i * 16
            got = plsc.load_gather(dom_v, [idx])
            vf_v[sl] = jnp.where(got == bvec, 1.0, 0.0).astype(jnp.float32)
            return carry

        lax.fori_loop(0, NB // 16, gath, 0)
        pltpu.sync_copy(vf_v, out_hbm)

    @pl.when(w == 0)
    def _():
        dedup(drugs_hbm, vfr_hbm)

    @pl.when(w == 1)
    def _():
        def zloop(i, carry):
            dmask_v[pl.ds(i * 16, 16)] = jnp.zeros((16,), jnp.float32)
            return carry

        lax.fori_loop(0, DPAD // 16, zloop, 0)
        pltpu.sync_copy(dis_hbm, all_v)

        def mscat(i, carry):
            idx = all_v[pl.ds(i * 16, 16)]
            plsc.store_scatter(dmask_v, [idx], jnp.ones((16,), jnp.float32))
            return carry

        lax.fori_loop(0, NB // 16, mscat, 0)
        pltpu.sync_copy(dmask_v, vfd_hbm)


def _gather_batch(g_r, g_d, er0, zr2, ed0, zd2, drugs, diseases):
    f = pl.kernel(
        _gather_body,
        out_type=[
            jax.ShapeDtypeStruct((NB, DIM), jnp.float32),
            jax.ShapeDtypeStruct((NB, DIM), jnp.float32),
            jax.ShapeDtypeStruct((NB, DIM), jnp.float32),
            jax.ShapeDtypeStruct((NB, DIM), jnp.float32),
            jax.ShapeDtypeStruct((NB,), jnp.float32),
            jax.ShapeDtypeStruct((DPAD,), jnp.float32),
        ],
        mesh=_sc_mesh(),
        scratch_types=[
            pltpu.VMEM((BPW,), jnp.int32),
            pltpu.VMEM((BPW, DIM), jnp.float32),
            pltpu.VMEM((BPW, DIM), jnp.float32),
            pltpu.VMEM((NB,), jnp.int32),
            pltpu.VMEM((RPAD,), jnp.int32),
            pltpu.VMEM((NB,), jnp.float32),
            pltpu.VMEM((DPAD,), jnp.float32),
            pltpu.SemaphoreType.DMA,
            pltpu.SemaphoreType.DMA,
        ],
        compiler_params=pltpu.CompilerParams(needs_layout_passes=False),
    )
    return f(g_r, g_d, er0, zr2, ed0, zd2, drugs, diseases)


_C1_BR = 1000


def _c1_body(er0_ref, zr1_ref, ed0_ref, zd1_ref, y_ref, x_ref):
    y_ref[...] = er0_ref[...] + zr1_ref[...]

    @pl.when(pl.program_id(0) == 0)
    def _():
        x_ref[...] = ed0_ref[...] + zd1_ref[...]


def _combine1(E_r_0, E_d_0, zr1p, zd1p):
    n = NDRUG // _C1_BR
    return pl.pallas_call(
        _c1_body,
        grid=(n,),
        in_specs=[
            pl.BlockSpec((_C1_BR, DIM), lambda i: (i, 0)),
            pl.BlockSpec((_C1_BR, DIM), lambda i: (i, 0)),
            pl.BlockSpec((NDIS, DIM), lambda i: (0, 0)),
            pl.BlockSpec((NDIS, DIM), lambda i: (0, 0)),
        ],
        out_specs=[
            pl.BlockSpec((_C1_BR, DIM), lambda i: (i, 0)),
            pl.BlockSpec((NDIS, DIM), lambda i: (0, 0)),
        ],
        out_shape=[
            jax.ShapeDtypeStruct((RPAD, DIM), jnp.float32),
            jax.ShapeDtypeStruct((DPAD, DIM), jnp.float32),
        ],
    )(E_r_0, zr1p, E_d_0, zd1p)


_M_BR = 400
_M_STEPS = NDRUG // _M_BR


def _m_body(rec_ref, x_ref, y_ref, er0_ref, ed0_ref,
            gr_ref, gd_ref, qacc_ref):
    i = pl.program_id(0)

    @pl.when(i == 0)
    def _():
        qacc_ref[...] = jnp.zeros_like(qacc_ref)

    rec_blk = rec_ref[...]
    gr_ref[...] = er0_ref[...] + jnp.dot(
        rec_blk, x_ref[...], preferred_element_type=jnp.float32)
    qacc_ref[...] += lax.dot_general(
        rec_blk, y_ref[...], (((0,), (0,)), ((), ())),
        preferred_element_type=jnp.float32)

    @pl.when(i == _M_STEPS - 1)
    def _():
        gd_ref[...] = ed0_ref[...] + qacc_ref[...]


def _matmul_pass(rec, X, Y, E_r_0, E_d_0):
    return pl.pallas_call(
        _m_body,
        grid=(_M_STEPS,),
        in_specs=[
            pl.BlockSpec((_M_BR, NDIS), lambda i: (i, 0)),
            pl.BlockSpec((NDIS, DIM), lambda i: (0, 0)),
            pl.BlockSpec((_M_BR, DIM), lambda i: (i, 0)),
            pl.BlockSpec((_M_BR, DIM), lambda i: (i, 0)),
            pl.BlockSpec((NDIS, DIM), lambda i: (0, 0)),
        ],
        out_specs=[
            pl.BlockSpec((_M_BR, DIM), lambda i: (i, 0)),
            pl.BlockSpec((NDIS, DIM), lambda i: (0, 0)),
        ],
        out_shape=[
            jax.ShapeDtypeStruct((NDRUG, DIM), jnp.float32),
            jax.ShapeDtypeStruct((NDIS, DIM), jnp.float32),
        ],
        scratch_shapes=[pltpu.VMEM((NDIS, DIM), jnp.float32)],
    )(rec, X, Y, E_r_0, E_d_0)


_S_BR = 256
_S_STEPS = NB // _S_BR


def _norm_rows(x):
    n = jnp.sqrt(jnp.sum(x * x, axis=1, keepdims=True))
    return x / jnp.maximum(n, 1e-12)


_D_BR = 400
_D_STEPS = NDIS // _D_BR


def _s_body(erb_ref, grb_ref, edb_ref, gdb_ref, grf_ref, gddom_ref,
            ed0f_ref, zd2f_ref, vfr_ref, vfdd_ref, vfdc_ref, labels_ref,
            p_ref, loss_ref, n2r_ref, n1d_ref, n2d_ref, acc_ref):
    i = pl.program_id(0)
    r0 = i * _S_BR

    @pl.when(i == 0)
    def _():
        n2r_ref[...] = _norm_rows(grf_ref[...])
        n1d_ref[...] = _norm_rows(ed0f_ref[...] + zd2f_ref[...])
        n2d_ref[...] = _norm_rows(gddom_ref[...])
        acc_ref[0] = 0.0
        acc_ref[1] = 0.0
        acc_ref[2] = 0.0

    vfr_row = vfr_ref[0, :]
    vfr_blk = vfr_ref[0, pl.ds(r0, _S_BR)]

    n1r = _norm_rows(erb_ref[...])
    n2r_blk = n2r_ref[pl.ds(r0, _S_BR), :]
    pos_r = jnp.sum(n1r * n2r_blk, axis=1)
    s_r = lax.dot_general(n1r, n2r_ref[...], (((1,), (1,)), ((), ())),
                          preferred_element_type=jnp.float32)
    all_r = jnp.sum(jnp.exp(s_r * INV_T) * vfr_row[None, :], axis=1)
    acc_ref[0] += jnp.sum(vfr_blk * (jnp.log(all_r) - pos_r * INV_T))

    @pl.when(i < _D_STEPS)
    def _():
        d0 = i * _D_BR
        md_row = vfdd_ref[0, pl.ds(0, NDIS)]
        md_blk = vfdc_ref[pl.ds(d0, _D_BR), 0]
        n1blk = n1d_ref[pl.ds(d0, _D_BR), :]
        pos_d = jnp.sum(n1blk * n2d_ref[pl.ds(d0, _D_BR), :], axis=1)
        s_d = lax.dot_general(n1blk, n2d_ref[...], (((1,), (1,)), ((), ())),
                              preferred_element_type=jnp.float32)
        all_d = jnp.sum(jnp.exp(s_d * INV_T) * md_row[None, :], axis=1)
        acc_ref[1] += jnp.sum(md_blk * (jnp.log(all_d) - pos_d * INV_T))

    demb = erb_ref[...] + grb_ref[...]
    hemb = edb_ref[...] + gdb_ref[...]
    scores = (WR * WD) * jnp.sum(demb * hemb, axis=1)
    p = jax.nn.sigmoid(scores)
    p_ref[0, :] = p
    pc = jnp.clip(p, 1e-7, 1.0 - 1e-7)
    lab = labels_ref[0, pl.ds(r0, _S_BR)]
    wgt = POS_W * lab + 1.0 - lab
    acc_ref[2] += jnp.sum(
        wgt * -(lab * jnp.log(pc) + (1.0 - lab) * jnp.log(1.0 - pc)))

    @pl.when(i == _S_STEPS - 1)
    def _():
        nvr = jnp.sum(vfr_row)
        nvd = jnp.sum(vfdd_ref[0, :])
        ssl = 0.05 * (acc_ref[0] / nvr) + 0.05 * (acc_ref[1] / nvd)
        loss_ref[...] = jnp.full((1, 1), acc_ref[2] / NB + 0.3 * ssl,
                                 jnp.float32)


def _ssl_bce(er_b, gr_b, ed_b, gd_b, G_d, E_d_0, zd2p, vfr, vfdd, vfdc,
             labels):
    return pl.pallas_call(
        _s_body,
        grid=(_S_STEPS,),
        in_specs=[
            pl.BlockSpec((_S_BR, DIM), lambda i: (i, 0)),
            pl.BlockSpec((_S_BR, DIM), lambda i: (i, 0)),
            pl.BlockSpec((_S_BR, DIM), lambda i: (i, 0)),
            pl.BlockSpec((_S_BR, DIM), lambda i: (i, 0)),
            pl.BlockSpec((NB, DIM), lambda i: (0, 0)),
            pl.BlockSpec((NDIS, DIM), lambda i: (0, 0)),
            pl.BlockSpec((NDIS, DIM), lambda i: (0, 0)),
            pl.BlockSpec((NDIS, DIM), lambda i: (0, 0)),
            pl.BlockSpec((1, NB), lambda i: (0, 0)),
            pl.BlockSpec((1, DPAD), lambda i: (0, 0)),
            pl.BlockSpec((DPAD, 1), lambda i: (0, 0)),
            pl.BlockSpec((1, NB), lambda i: (0, 0)),
        ],
        out_specs=[
            pl.BlockSpec((1, _S_BR), lambda i: (0, i)),
            pl.BlockSpec((1, 1), lambda i: (0, 0)),
        ],
        out_shape=[
            jax.ShapeDtypeStruct((1, NB), jnp.float32),
            jax.ShapeDtypeStruct((1, 1), jnp.float32),
        ],
        scratch_shapes=[
            pltpu.VMEM((NB, DIM), jnp.float32),
            pltpu.VMEM((NDIS, DIM), jnp.float32),
            pltpu.VMEM((NDIS, DIM), jnp.float32),
            pltpu.SMEM((4,), jnp.float32),
        ],
    )(er_b, gr_b, ed_b, gd_b, gr_b, G_d, E_d_0, zd2p, vfr, vfdd, vfdc,
      labels)


def kernel(drugs, diseases, labels, E_r_0, E_d_0, adj_idx, adj_val, rec):
    src = adj_idx[0]
    dst = adj_idx[1]
    padn = EPAD - NE
    src_p = jnp.concatenate([src, jnp.zeros((padn,), jnp.int32)])
    dst_p = jnp.concatenate([dst, jnp.zeros((padn,), jnp.int32)])
    val_p = jnp.concatenate([adj_val, jnp.zeros((padn,), jnp.float32)])
    src_p = src_p.reshape(NS * NCH, CHUNK)
    dst_p = dst_p.reshape(NS * NCH, CHUNK)
    zeros_rows = jnp.zeros((RSTRIPE, DIM), jnp.float32)
    Er0p = jnp.concatenate(
        [E_r_0, jnp.zeros((RPAD - NDRUG, DIM), jnp.float32)])
    Ed0p = jnp.concatenate(
        [E_d_0, jnp.zeros((DPAD - NDIS, DIM), jnp.float32)])

    zr1p, zd1p = _edge_pass(src_p, dst_p, val_p, Er0p, Ed0p, zeros_rows)
    Y, X = _combine1(E_r_0, E_d_0, zr1p, zd1p)

    zr2p, zd2p = _edge_pass(src_p, dst_p, val_p, Y, X, zeros_rows)
    G_r, G_d = _matmul_pass(rec, X, Y, E_r_0, E_d_0)

    er_b, gr_b, ed_b, gd_b, vfr, vfd = _gather_batch(
        G_r, G_d, E_r_0, zr2p, E_d_0, zd2p, drugs, diseases)

    p2, loss2 = _ssl_bce(er_b, gr_b, ed_b, gd_b, G_d, E_d_0, zd2p,
                         vfr.reshape(1, NB), vfd.reshape(1, DPAD),
                         vfd.reshape(DPAD, 1), labels.reshape(1, NB))
    return (loss2[0, 0], p2.reshape(NB))
